# baseline (speedup 1.0000x reference)
import functools

import jax
import jax.numpy as jnp
from jax import lax
from jax.experimental import pallas as pl
from jax.experimental.pallas import tpu as pltpu
from jax.experimental.pallas import tpu_sc as plsc

K_NN = 64
_CHUNK = 2048
_BUCKET = 256
_PK = _BUCKET // 2
_BPC = _CHUNK // _BUCKET
_NEG = float("-inf")
_NC, _NS, _L = 2, 16, 16
_NW = _NC * _NS


def _f2key(x):
    b = lax.bitcast_convert_type(x, jnp.int32)
    mask = lax.shift_right_arithmetic(b, 31) & jnp.int32(0x7FFFFFFF)
    s = lax.bitwise_xor(b, mask)
    return lax.bitcast_convert_type(s, jnp.uint32) + jnp.uint32(0x80000000)


def _key2f(u):
    s = lax.bitcast_convert_type(u + jnp.uint32(0x80000000), jnp.int32)
    mask = lax.shift_right_arithmetic(s, 31) & jnp.int32(0x7FFFFFFF)
    b = lax.bitwise_xor(s, mask)
    return lax.bitcast_convert_type(b, jnp.float32)


def _sims_kernel(feat_ref, glo_ref, ghi_ref, pk_ref, bmax_ref, *, n_real):
    j = pl.program_id(0)
    q = feat_ref[...]
    qn = (q * lax.rsqrt(jnp.maximum(jnp.sum(q * q, axis=1, keepdims=True),
                                    1e-30))).astype(jnp.bfloat16)
    nq = q.shape[0]

    def norm16(g_ref):
        g = g_ref[...].reshape(_CHUNK // 2, q.shape[1])
        return (g * lax.rsqrt(jnp.maximum(
            jnp.sum(g * g, axis=1, keepdims=True), 1e-30))).astype(jnp.bfloat16)

    gn_lo = norm16(glo_ref)
    gn_hi = norm16(ghi_ref)
    li = lax.broadcasted_iota(jnp.int32, (nq, _PK), 1)
    bms = []
    for b in range(_BPC):
        def half(gn, col0):
            s = lax.dot_general(qn, gn[b * _PK:(b + 1) * _PK, :],
                                (((1,), (1,)), ((), ())),
                                preferred_element_type=jnp.float32)
            s = jnp.where(li + col0 < n_real, s, _NEG)
            s16 = s.astype(jnp.bfloat16)
            sf = s16.astype(jnp.float32)
            bm = jnp.max(sf, axis=1, keepdims=True)
            u = lax.bitcast_convert_type(s16, jnp.uint16).astype(jnp.uint32)
            return u, bm

        c0 = j * _CHUNK + b * _BUCKET
        u_lo, bm_lo = half(gn_lo, c0)
        u_hi, bm_hi = half(gn_hi, c0 + _PK)
        pk = lax.shift_left(u_hi, jnp.uint32(16)) | u_lo
        pk_ref[:, b, :] = lax.bitcast_convert_type(pk, jnp.int32)
        bms.append(jnp.maximum(bm_lo, bm_hi))
    bmax_ref[...] = jnp.concatenate(bms, axis=1).reshape(1, nq, _BPC)


def _sc_select(sims2d, bmax, out, bmax_va, bmax_vb, ids_v, idx_va, idx_vb,
               cand_va, cand_vb, surv_v, orow_v, bsa, bsb, gsa, gsb, sem,
               *, nq, nb_rows, nb_pad):
    wid = lax.axis_index("s") * _NC + lax.axis_index("c")
    rows_per_w = nq // _NW
    base = wid * rows_per_w
    nbv = nb_pad // _L
    iota = lax.broadcasted_iota(jnp.int32, (_L,), 0)
    ninf = jnp.full((_L,), _NEG, jnp.float32)
    zf = jnp.zeros((_L,), jnp.float32)

    def popcnt(msk):
        return plsc.all_reduce_population_count(msk)

    def to_scalar_i(vec):
        return vec[0]

    def cnt_ge_static(ref, nvreg, t):
        cs = [jnp.zeros((_L,), jnp.int32) for _ in range(4)]
        for k in range(nvreg):
            cs[k % 4] = cs[k % 4] + popcnt(ref[pl.ds(k * _L, _L)] >= t)
        return (cs[0] + cs[1]) + (cs[2] + cs[3])

    def bis_while(cnt_fn, lo0, hi0):
        def cond(carry):
            lo, hi, it = carry
            return jnp.logical_and(it < 32, (hi - lo)[0] > 1)
        def body(carry):
            lo, hi, it = carry
            mid = lo + lax.shift_right_logical(hi - lo, jnp.uint32(1))
            ge = cnt_fn(_key2f(mid)) >= K_NN
            return jnp.where(ge, mid, lo), jnp.where(ge, hi, mid), it + 1
        lo, _, _ = lax.while_loop(cond, body, (lo0, hi0, jnp.int32(0)))
        return _key2f(lo)

    def a_phase(i, bmax_v, idx_v, bsem, gsem, cand_v):
        r = base + i
        pltpu.make_async_copy(bmax.at[r], bmax_v, bsem).wait()

        mxs = [ninf, ninf]
        mns = [jnp.full((_L,), jnp.inf, jnp.float32) for _ in range(2)]
        for k in range(nbv - 1):
            v = bmax_v[pl.ds(k * _L, _L)]
            mxs[k % 2] = jnp.maximum(mxs[k % 2], v)
            mns[k % 2] = jnp.minimum(mns[k % 2], v)
        mvec = jnp.maximum(jnp.maximum(mxs[0], mxs[1]),
                           bmax_v[pl.ds((nbv - 1) * _L, _L)])
        nvec = jnp.minimum(mns[0], mns[1])
        m = mvec[0]
        lo_f = nvec[0]
        for k in range(1, _L):
            m = jnp.maximum(m, mvec[k])
            lo_f = jnp.minimum(lo_f, nvec[k])
        m_v = jnp.full((_L,), 0.0, jnp.float32) + m
        lo_v = jnp.full((_L,), 0.0, jnp.float32) + lo_f

        t_low = bis_while(lambda t: cnt_ge_static(bmax_v, nbv, t),
                          _f2key(lo_v), _f2key(m_v) + jnp.uint32(1))

        def gt_body(k, off):
            v = bmax_v[pl.ds(k * _L, _L)]
            msk = v > t_low
            plsc.store_compressed(ids_v.at[pl.ds(off, _L)], k * _L + iota,
                                  mask=msk)
            return off + to_scalar_i(popcnt(msk))

        off = lax.fori_loop(0, nbv, gt_body, jnp.int32(0), unroll=4)

        def eq_body(k, off):
            v = bmax_v[pl.ds(k * _L, _L)]
            msk = v == t_low
            @pl.when(off < K_NN)
            def _():
                plsc.store_compressed(ids_v.at[pl.ds(off, _L)],
                                      k * _L + iota, mask=msk)
            return off + to_scalar_i(popcnt(msk))

        lax.fori_loop(0, nbv, eq_body, off, unroll=4)

        for k in range(K_NN // _L):
            idx_v[pl.ds(k * _L, _L)] = ids_v[pl.ds(k * _L, _L)] + r * nb_rows
        pltpu.async_copy(sims2d.at[idx_v], cand_v, gsem)

        @pl.when(i + 2 < rows_per_w)
        def _():
            pltpu.async_copy(bmax.at[r + 2], bmax_v, bsem)
        return t_low, m_v

    def b_phase(i, cand_v, gsem, t_low, m_v):
        r = base + i
        pltpu.make_async_copy(sims2d.at[pl.ds(0, K_NN)], cand_v, gsem).wait()

        himask = jnp.full((_L,), -65536, jnp.int32)
        def f_body(k, offs):
            j2 = k >> 3
            lb = (k & 7) * _L
            new = list(offs)
            for g in range(4):
                w = cand_v[g * (K_NN // 4) + j2, pl.ds(lb, _L)]
                f_lo = lax.bitcast_convert_type(
                    lax.shift_left(w, jnp.int32(16)), jnp.float32)
                f_hi = lax.bitcast_convert_type(w & himask, jnp.float32)
                for h, fv in ((0, f_lo), (1, f_hi)):
                    sg = 2 * g + h
                    msk = fv >= t_low
                    plsc.store_compressed(surv_v.at[sg, pl.ds(new[sg], _L)],
                                          fv, mask=msk)
                    new[sg] = new[sg] + to_scalar_i(popcnt(msk))
            return tuple(new)

        z8 = (jnp.int32(0),) * 8
        offs = lax.fori_loop(0, (K_NN // 4) * (_PK // _L), f_body, z8,
                             unroll=4)
        for g in range(8):
            surv_v[g, pl.ds(offs[g], _L)] = ninf
        nvs = [lax.div(offs[g] + (_L - 1), jnp.int32(_L)) for g in range(8)]

        def cnt8(t):
            c = jnp.zeros((_L,), jnp.int32)
            for g in range(8):
                def b(k, c2):
                    return c2 + popcnt(surv_v[g, pl.ds(k * _L, _L)] >= t)
                c = lax.fori_loop(0, nvs[g], b, c)
            return c

        v64 = bis_while(cnt8, _f2key(t_low), _f2key(m_v) + jnp.uint32(1))

        def s_body_g(g):
            def s_body(k, carry):
                zv, tv, cv = carry
                v = surv_v[g, pl.ds(k * _L, _L)]
                gt = v > v64
                e = jnp.exp(v - m_v)
                zv = zv + jnp.where(gt, e, 0.0)
                tv = tv + jnp.where(gt, (v - m_v) * e, 0.0)
                cv = cv + popcnt(gt)
                return zv, tv, cv
            return s_body

        carry = (zf, zf, jnp.zeros((_L,), jnp.int32))
        for g in range(8):
            carry = lax.fori_loop(0, nvs[g], s_body_g(g), carry)
        zv, tv, cv = carry
        z_gt = zv[0]
        t_gt = tv[0]
        for k in range(1, _L):
            z_gt = z_gt + zv[k]
            t_gt = t_gt + tv[k]

        orow_v[...] = ((iota == 0).astype(jnp.float32) * z_gt
                       + (iota == 1).astype(jnp.float32) * t_gt
                       + (iota == 2).astype(jnp.float32) * cv.astype(jnp.float32)
                       + (iota == 3).astype(jnp.float32) * v64
                       + (iota == 4).astype(jnp.float32) * m_v[0])
        pltpu.sync_copy(orow_v, out.at[r])

    pltpu.async_copy(bmax.at[base], bmax_va, bsa)
    pltpu.async_copy(bmax.at[base + 1], bmax_vb, bsb)

    def pair_body(r2, carry):
        t_prev, m_prev = carry
        t_a, m_a = a_phase(2 * r2, bmax_va, idx_va, bsa, gsa, cand_va)

        @pl.when(r2 >= 1)
        def _():
            b_phase(2 * r2 - 1, cand_vb, gsb, t_prev, m_prev)

        t_b, m_b = a_phase(2 * r2 + 1, bmax_vb, idx_vb, bsb, gsb, cand_vb)
        b_phase(2 * r2, cand_va, gsa, t_a, m_a)
        return t_b, m_b

    t_fin, m_fin = lax.fori_loop(0, rows_per_w // 2, pair_body, (zf, zf))
    b_phase(rows_per_w - 1, cand_vb, gsb, t_fin, m_fin)


def _finish_kernel(st_ref, out_ref, *, nq):
    x = st_ref[...]
    z_gt, t_gt = x[:, 0:1], x[:, 1:2]
    c_gt, v64, m = x[:, 2:3], x[:, 3:4], x[:, 4:5]
    n_tie = jnp.float32(K_NN) - c_gt
    ut = v64 - m
    et = jnp.exp(ut)
    z = z_gt + n_tie * et
    t = t_gt + n_tie * ut * et
    ent = jnp.log(z) - t / z
    out_ref[...] = jnp.sum(ent).reshape(1, 1) * (1.0 / nq)


def kernel(feat, gallery_features):
    nq, d = feat.shape
    ng = gallery_features.shape[0]
    ng_pad = ((ng + _CHUNK - 1) // _CHUNK) * _CHUNK
    nb_rows = ng_pad // _BUCKET
    nb_pad = ((nb_rows + _L - 1) // _L) * _L
    nchunks = ng_pad // _CHUNK
    gal = jnp.pad(gallery_features, ((0, ng_pad - ng), (0, 0)))
    gal4 = gal.reshape(nb_rows, 2, _PK, d)

    def run_half(feat_h):
        nqh = feat_h.shape[0]
        pk, bmax3 = pl.pallas_call(
            functools.partial(_sims_kernel, n_real=ng),
            grid=(nchunks,),
            in_specs=[
                pl.BlockSpec((nqh, d), lambda j: (0, 0)),
                pl.BlockSpec((_BPC, 1, _PK, d), lambda j: (j, 0, 0, 0)),
                pl.BlockSpec((_BPC, 1, _PK, d), lambda j: (j, 1, 0, 0)),
            ],
            out_specs=[
                pl.BlockSpec((nqh, _BPC, _PK), lambda j: (0, j, 0)),
                pl.BlockSpec((1, nqh, _BPC), lambda j: (j, 0, 0)),
            ],
            out_shape=[
                jax.ShapeDtypeStruct((nqh, nb_rows, _PK), jnp.int32),
                jax.ShapeDtypeStruct((nchunks, nqh, _BPC), jnp.float32),
            ],
        )(feat_h, gal4, gal4)

        bmax = jnp.pad(bmax3.transpose(1, 0, 2).reshape(nqh, nb_rows),
                       ((0, 0), (0, nb_pad - nb_rows)), constant_values=_NEG)
        sims2d = pk.reshape(nqh * nb_rows, _PK)

        sc_fn = functools.partial(
            pl.kernel,
            mesh=plsc.VectorSubcoreMesh(core_axis_name="c",
                                        subcore_axis_name="s"),
            compiler_params=pltpu.CompilerParams(needs_layout_passes=False),
            out_type=jax.ShapeDtypeStruct((nqh, _L), jnp.float32),
            scratch_types=[
                pltpu.VMEM((nb_pad,), jnp.float32),
                pltpu.VMEM((nb_pad,), jnp.float32),
                pltpu.VMEM((K_NN + 2 * _L,), jnp.int32),
                pltpu.VMEM((K_NN,), jnp.int32),
                pltpu.VMEM((K_NN,), jnp.int32),
                pltpu.VMEM((K_NN, _PK), jnp.int32),
                pltpu.VMEM((K_NN, _PK), jnp.int32),
                pltpu.VMEM((8, K_NN * _PK // 4 + _L), jnp.float32),
                pltpu.VMEM((_L,), jnp.float32),
                pltpu.SemaphoreType.DMA,
                pltpu.SemaphoreType.DMA,
                pltpu.SemaphoreType.DMA,
                pltpu.SemaphoreType.DMA,
                pltpu.SemaphoreType.DMA,
            ],
        )(functools.partial(_sc_select, nq=nqh, nb_rows=nb_rows,
                            nb_pad=nb_pad))
        return sc_fn(sims2d, bmax)

    h = nq // 8
    stats = jnp.concatenate(
        [run_half(feat[i * h:(i + 1) * h]) for i in range(8)], axis=0)

    out = pl.pallas_call(
        functools.partial(_finish_kernel, nq=float(nq)),
        in_specs=[pl.BlockSpec((nq, _L), lambda: (0, 0))],
        out_specs=pl.BlockSpec((1, 1), lambda: (0, 0)),
        out_shape=jax.ShapeDtypeStruct((1, 1), jnp.float32),
    )(stats)
    return out[0, 0]


# speedup vs baseline: 1.2221x; 1.2221x over previous
import functools

import jax
import jax.numpy as jnp
from jax import lax
from jax.experimental import pallas as pl
from jax.experimental.pallas import tpu as pltpu
from jax.experimental.pallas import tpu_sc as plsc

K_NN = 64
_CHUNK = 2048
_BUCKET = 256
_PK = _BUCKET // 2
_BPC = _CHUNK // _BUCKET
_NEG = float("-inf")
_NC, _NS, _L = 2, 16, 16
_NW = _NC * _NS


def _f2key(x):
    b = lax.bitcast_convert_type(x, jnp.int32)
    mask = lax.shift_right_arithmetic(b, 31) & jnp.int32(0x7FFFFFFF)
    s = lax.bitwise_xor(b, mask)
    return lax.bitcast_convert_type(s, jnp.uint32) + jnp.uint32(0x80000000)


def _key2f(u):
    s = lax.bitcast_convert_type(u + jnp.uint32(0x80000000), jnp.int32)
    mask = lax.shift_right_arithmetic(s, 31) & jnp.int32(0x7FFFFFFF)
    b = lax.bitwise_xor(s, mask)
    return lax.bitcast_convert_type(b, jnp.float32)


def _sims_kernel(feat_ref, glo_ref, ghi_ref, pk_ref, bmax_ref, *, n_real):
    j = pl.program_id(0)
    q = feat_ref[...]
    qn = (q * lax.rsqrt(jnp.maximum(jnp.sum(q * q, axis=1, keepdims=True),
                                    1e-30))).astype(jnp.bfloat16)
    nq = q.shape[0]

    def norm16(g_ref):
        g = g_ref[...].reshape(_CHUNK // 2, q.shape[1])
        return (g * lax.rsqrt(jnp.maximum(
            jnp.sum(g * g, axis=1, keepdims=True), 1e-30))).astype(jnp.bfloat16)

    gn_lo = norm16(glo_ref)
    gn_hi = norm16(ghi_ref)
    li = lax.broadcasted_iota(jnp.int32, (nq, _PK), 1)
    bms = []
    for b in range(_BPC):
        def half(gn, col0):
            s = lax.dot_general(qn, gn[b * _PK:(b + 1) * _PK, :],
                                (((1,), (1,)), ((), ())),
                                preferred_element_type=jnp.float32)
            s = jnp.where(li + col0 < n_real, s, _NEG)
            s16 = s.astype(jnp.bfloat16)
            sf = s16.astype(jnp.float32)
            bm = jnp.max(sf, axis=1, keepdims=True)
            u = lax.bitcast_convert_type(s16, jnp.uint16).astype(jnp.uint32)
            return u, bm

        c0 = j * _CHUNK + b * _BUCKET
        u_lo, bm_lo = half(gn_lo, c0)
        u_hi, bm_hi = half(gn_hi, c0 + _PK)
        pk = lax.shift_left(u_hi, jnp.uint32(16)) | u_lo
        pk_ref[:, b, :] = lax.bitcast_convert_type(pk, jnp.int32)
        bms.append(jnp.maximum(bm_lo, bm_hi))
    bmax_ref[...] = jnp.concatenate(bms, axis=1).reshape(1, nq, _BPC)


def _sc_select(sims2d, bmax, out, bmax_va, bmax_vb, ids_v, idx_va, idx_vb,
               cand_va, cand_vb, surv_v, orow_v, bsa, bsb, gsa, gsb, sem,
               *, nq, nb_rows, nb_pad):
    wid = lax.axis_index("s") * _NC + lax.axis_index("c")
    rows_per_w = nq // _NW
    base = wid * rows_per_w
    nbv = nb_pad // _L
    iota = lax.broadcasted_iota(jnp.int32, (_L,), 0)
    ninf = jnp.full((_L,), _NEG, jnp.float32)
    zf = jnp.zeros((_L,), jnp.float32)

    def popcnt(msk):
        return plsc.all_reduce_population_count(msk)

    def to_scalar_i(vec):
        return vec[0]

    def cnt_ge_static(ref, nvreg, t):
        cs = [jnp.zeros((_L,), jnp.int32) for _ in range(4)]
        for k in range(nvreg):
            cs[k % 4] = cs[k % 4] + popcnt(ref[pl.ds(k * _L, _L)] >= t)
        return (cs[0] + cs[1]) + (cs[2] + cs[3])

    def bis_while(cnt_fn, lo0, hi0):
        def cond(carry):
            lo, hi, it = carry
            return jnp.logical_and(it < 32, (hi - lo)[0] > 1)
        def body(carry):
            lo, hi, it = carry
            mid = lo + lax.shift_right_logical(hi - lo, jnp.uint32(1))
            ge = cnt_fn(_key2f(mid)) >= K_NN
            return jnp.where(ge, mid, lo), jnp.where(ge, hi, mid), it + 1
        lo, _, _ = lax.while_loop(cond, body, (lo0, hi0, jnp.int32(0)))
        return _key2f(lo)

    def a_phase(i, bmax_v, idx_v, bsem, gsem, cand_v):
        r = base + i
        pltpu.make_async_copy(bmax.at[r], bmax_v, bsem).wait()

        mxs = [ninf, ninf]
        mns = [jnp.full((_L,), jnp.inf, jnp.float32) for _ in range(2)]
        for k in range(nbv - 1):
            v = bmax_v[pl.ds(k * _L, _L)]
            mxs[k % 2] = jnp.maximum(mxs[k % 2], v)
            mns[k % 2] = jnp.minimum(mns[k % 2], v)
        mvec = jnp.maximum(jnp.maximum(mxs[0], mxs[1]),
                           bmax_v[pl.ds((nbv - 1) * _L, _L)])
        nvec = jnp.minimum(mns[0], mns[1])
        m = mvec[0]
        lo_f = nvec[0]
        for k in range(1, _L):
            m = jnp.maximum(m, mvec[k])
            lo_f = jnp.minimum(lo_f, nvec[k])
        m_v = jnp.full((_L,), 0.0, jnp.float32) + m
        lo_v = jnp.full((_L,), 0.0, jnp.float32) + lo_f

        t_low = bis_while(lambda t: cnt_ge_static(bmax_v, nbv, t),
                          _f2key(lo_v), _f2key(m_v) + jnp.uint32(1))

        def gt_body(k, off):
            v = bmax_v[pl.ds(k * _L, _L)]
            msk = v > t_low
            plsc.store_compressed(ids_v.at[pl.ds(off, _L)], k * _L + iota,
                                  mask=msk)
            return off + to_scalar_i(popcnt(msk))

        off = lax.fori_loop(0, nbv, gt_body, jnp.int32(0), unroll=4)

        def eq_body(k, off):
            v = bmax_v[pl.ds(k * _L, _L)]
            msk = v == t_low
            @pl.when(off < K_NN)
            def _():
                plsc.store_compressed(ids_v.at[pl.ds(off, _L)],
                                      k * _L + iota, mask=msk)
            return off + to_scalar_i(popcnt(msk))

        lax.fori_loop(0, nbv, eq_body, off, unroll=4)

        for k in range(K_NN // _L):
            idx_v[pl.ds(k * _L, _L)] = ids_v[pl.ds(k * _L, _L)] + r * nb_rows
        pltpu.async_copy(sims2d.at[idx_v], cand_v, gsem)

        @pl.when(i + 2 < rows_per_w)
        def _():
            pltpu.async_copy(bmax.at[r + 2], bmax_v, bsem)
        return t_low, m_v

    def b_phase(i, cand_v, gsem, t_low, m_v):
        r = base + i
        pltpu.make_async_copy(sims2d.at[pl.ds(0, K_NN)], cand_v, gsem).wait()

        himask = jnp.full((_L,), -65536, jnp.int32)
        def f_body(k, offs):
            j2 = k >> 3
            lb = (k & 7) * _L
            new = list(offs)
            for g in range(4):
                w = cand_v[g * (K_NN // 4) + j2, pl.ds(lb, _L)]
                f_lo = lax.bitcast_convert_type(
                    lax.shift_left(w, jnp.int32(16)), jnp.float32)
                f_hi = lax.bitcast_convert_type(w & himask, jnp.float32)
                for h, fv in ((0, f_lo), (1, f_hi)):
                    sg = 2 * g + h
                    msk = fv >= t_low
                    plsc.store_compressed(surv_v.at[sg, pl.ds(new[sg], _L)],
                                          fv, mask=msk)
                    new[sg] = new[sg] + to_scalar_i(popcnt(msk))
            return tuple(new)

        z8 = (jnp.int32(0),) * 8
        offs = lax.fori_loop(0, (K_NN // 4) * (_PK // _L), f_body, z8,
                             unroll=4)
        for g in range(8):
            surv_v[g, pl.ds(offs[g], _L)] = ninf
        nvs = [lax.div(offs[g] + (_L - 1), jnp.int32(_L)) for g in range(8)]

        def cnt8(t):
            c = jnp.zeros((_L,), jnp.int32)
            for g in range(8):
                def b(k, c2):
                    return c2 + popcnt(surv_v[g, pl.ds(k * _L, _L)] >= t)
                c = lax.fori_loop(0, nvs[g], b, c)
            return c

        v64 = bis_while(cnt8, _f2key(t_low), _f2key(m_v) + jnp.uint32(1))

        def s_body_g(g):
            def s_body(k, carry):
                zv, tv, cv = carry
                v = surv_v[g, pl.ds(k * _L, _L)]
                gt = v > v64
                e = jnp.exp(v - m_v)
                zv = zv + jnp.where(gt, e, 0.0)
                tv = tv + jnp.where(gt, (v - m_v) * e, 0.0)
                cv = cv + popcnt(gt)
                return zv, tv, cv
            return s_body

        carry = (zf, zf, jnp.zeros((_L,), jnp.int32))
        for g in range(8):
            carry = lax.fori_loop(0, nvs[g], s_body_g(g), carry)
        zv, tv, cv = carry
        z_gt = zv[0]
        t_gt = tv[0]
        for k in range(1, _L):
            z_gt = z_gt + zv[k]
            t_gt = t_gt + tv[k]

        orow_v[...] = ((iota == 0).astype(jnp.float32) * z_gt
                       + (iota == 1).astype(jnp.float32) * t_gt
                       + (iota == 2).astype(jnp.float32) * cv.astype(jnp.float32)
                       + (iota == 3).astype(jnp.float32) * v64
                       + (iota == 4).astype(jnp.float32) * m_v[0])
        pltpu.sync_copy(orow_v, out.at[r])

    pltpu.async_copy(bmax.at[base], bmax_va, bsa)
    pltpu.async_copy(bmax.at[base + 1], bmax_vb, bsb)

    def pair_body(r2, carry):
        t_prev, m_prev = carry
        t_a, m_a = a_phase(2 * r2, bmax_va, idx_va, bsa, gsa, cand_va)

        @pl.when(r2 >= 1)
        def _():
            b_phase(2 * r2 - 1, cand_vb, gsb, t_prev, m_prev)

        t_b, m_b = a_phase(2 * r2 + 1, bmax_vb, idx_vb, bsb, gsb, cand_vb)
        b_phase(2 * r2, cand_va, gsa, t_a, m_a)
        return t_b, m_b

    t_fin, m_fin = lax.fori_loop(0, rows_per_w // 2, pair_body, (zf, zf))
    b_phase(rows_per_w - 1, cand_vb, gsb, t_fin, m_fin)


def _finish_kernel(st_ref, out_ref, *, nq):
    x = st_ref[...]
    z_gt, t_gt = x[:, 0:1], x[:, 1:2]
    c_gt, v64, m = x[:, 2:3], x[:, 3:4], x[:, 4:5]
    n_tie = jnp.float32(K_NN) - c_gt
    ut = v64 - m
    et = jnp.exp(ut)
    z = z_gt + n_tie * et
    t = t_gt + n_tie * ut * et
    ent = jnp.log(z) - t / z
    out_ref[...] = jnp.sum(ent).reshape(1, 1) * (1.0 / nq)


def kernel(feat, gallery_features):
    nq, d = feat.shape
    ng = gallery_features.shape[0]
    ng_pad = ((ng + _CHUNK - 1) // _CHUNK) * _CHUNK
    nb_rows = ng_pad // _BUCKET
    nb_pad = ((nb_rows + _L - 1) // _L) * _L
    nchunks = ng_pad // _CHUNK
    gal = jnp.pad(gallery_features, ((0, ng_pad - ng), (0, 0)))
    gal4 = gal.reshape(nb_rows, 2, _PK, d)

    def run_half(feat_h):
        nqh = feat_h.shape[0]
        pk, bmax3 = pl.pallas_call(
            functools.partial(_sims_kernel, n_real=ng),
            grid=(nchunks,),
            in_specs=[
                pl.BlockSpec((nqh, d), lambda j: (0, 0)),
                pl.BlockSpec((_BPC, 1, _PK, d), lambda j: (j, 0, 0, 0)),
                pl.BlockSpec((_BPC, 1, _PK, d), lambda j: (j, 1, 0, 0)),
            ],
            out_specs=[
                pl.BlockSpec((nqh, _BPC, _PK), lambda j: (0, j, 0)),
                pl.BlockSpec((1, nqh, _BPC), lambda j: (j, 0, 0)),
            ],
            out_shape=[
                jax.ShapeDtypeStruct((nqh, nb_rows, _PK), jnp.int32),
                jax.ShapeDtypeStruct((nchunks, nqh, _BPC), jnp.float32),
            ],
        )(feat_h, gal4, gal4)

        bmax = jnp.pad(bmax3.transpose(1, 0, 2).reshape(nqh, nb_rows),
                       ((0, 0), (0, nb_pad - nb_rows)), constant_values=_NEG)
        sims2d = pk.reshape(nqh * nb_rows, _PK)

        sc_fn = functools.partial(
            pl.kernel,
            mesh=plsc.VectorSubcoreMesh(core_axis_name="c",
                                        subcore_axis_name="s"),
            compiler_params=pltpu.CompilerParams(needs_layout_passes=False),
            out_type=jax.ShapeDtypeStruct((nqh, _L), jnp.float32),
            scratch_types=[
                pltpu.VMEM((nb_pad,), jnp.float32),
                pltpu.VMEM((nb_pad,), jnp.float32),
                pltpu.VMEM((K_NN + 2 * _L,), jnp.int32),
                pltpu.VMEM((K_NN,), jnp.int32),
                pltpu.VMEM((K_NN,), jnp.int32),
                pltpu.VMEM((K_NN, _PK), jnp.int32),
                pltpu.VMEM((K_NN, _PK), jnp.int32),
                pltpu.VMEM((8, K_NN * _PK // 4 + _L), jnp.float32),
                pltpu.VMEM((_L,), jnp.float32),
                pltpu.SemaphoreType.DMA,
                pltpu.SemaphoreType.DMA,
                pltpu.SemaphoreType.DMA,
                pltpu.SemaphoreType.DMA,
                pltpu.SemaphoreType.DMA,
            ],
        )(functools.partial(_sc_select, nq=nqh, nb_rows=nb_rows,
                            nb_pad=nb_pad))
        return sc_fn(sims2d, bmax)

    h = nq // 4
    stats = jnp.concatenate(
        [run_half(feat[i * h:(i + 1) * h]) for i in range(4)], axis=0)

    out = pl.pallas_call(
        functools.partial(_finish_kernel, nq=float(nq)),
        in_specs=[pl.BlockSpec((nq, _L), lambda: (0, 0))],
        out_specs=pl.BlockSpec((1, 1), lambda: (0, 0)),
        out_shape=jax.ShapeDtypeStruct((1, 1), jnp.float32),
    )(stats)
    return out[0, 0]
